# Initial kernel scaffold; baseline (speedup 1.0000x reference)
import jax
import jax.numpy as jnp
from jax.experimental import pallas as pl


def kernel(features, node_order_bottomup, adjacency_list, edge_order_bottomup, vocabs, tree_sizes, res_table, leaf_table, W_ih, b_ih, b_hh, W_iou, b_iou, U_iou, W_f, b_f, U_f, Wm, bm, Wv, bv):
    raise NotImplementedError("write your pallas kernel here")


# speedup vs baseline: 36.2975x; 36.2975x over previous
import functools

import numpy as np
import jax
import jax.numpy as jnp
from jax import lax
from jax.experimental import pallas as pl
from jax.experimental.pallas import tpu as pltpu
from jax.experimental.pallas import tpu_sc as plsc

H = 256
DEPTH = 12
T = 24
S = 2 ** DEPTH - 1
LEAVES = 2 ** (DEPTH - 1)
LATENT = 64
NW = 32
B_GATHER = T * LEAVES


def _build_perms():
    parts = []
    for l in range(DEPTH - 1):
        js = np.arange(2 ** l - 1, 2 ** (l + 1) - 1)
        parts.append((np.arange(T)[:, None] * S + js[None, :]).reshape(-1))
    int_perm = np.concatenate(parts)
    int_perm = np.concatenate(
        [int_perm, np.zeros(B_GATHER - int_perm.size, np.int64)])
    leaf_perm = (np.arange(T)[:, None] * S
                 + np.arange(LEAVES - 1, S)[None, :]).reshape(-1)
    return jnp.asarray(int_perm, jnp.int32), jnp.asarray(leaf_perm, jnp.int32)


_INT_PERM, _LEAF_PERM = _build_perms()


def _tables_body(leaf_t_ref, res_t_ref, w_ih_ref, b_ihh_ref, w_iou_ref,
                 b_iou_ref, w_f_ref, b_f_ref, hc_ref, xwiou_ref, xwf_ref):
    gates = lax.dot_general(
        leaf_t_ref[...], w_ih_ref[...], (((1,), (1,)), ((), ())),
        preferred_element_type=jnp.float32) + b_ihh_ref[...]
    c = jax.nn.sigmoid(gates[:, 0:H]) * jnp.tanh(gates[:, 2 * H:3 * H])
    h = jax.nn.sigmoid(gates[:, 3 * H:4 * H]) * jnp.tanh(c)
    hc_ref[:, 0:H] = h
    hc_ref[:, H:2 * H] = c
    res = res_t_ref[...]
    xwiou_ref[...] = jnp.dot(
        res, w_iou_ref[...], preferred_element_type=jnp.float32) + b_iou_ref[...]
    xwf_ref[...] = jnp.dot(
        res, w_f_ref[...], preferred_element_type=jnp.float32) + b_f_ref[...]


def _tables_call(leaf_table, res_table, W_ih, b_ihh, W_iou, b_iou, W_f, b_f):
    V = leaf_table.shape[0]
    return pl.pallas_call(
        _tables_body,
        out_shape=[
            jax.ShapeDtypeStruct((V, 2 * H), jnp.float32),
            jax.ShapeDtypeStruct((V, 3 * H), jnp.float32),
            jax.ShapeDtypeStruct((V, H), jnp.float32),
        ],
    )(leaf_table, res_table, W_ih, b_ihh, W_iou, b_iou, W_f, b_f)


def _sc_gather(table, idx):
    V, D = table.shape
    B = idx.shape[0]
    b_per_w = B // NW
    C = {256: 128, 512: 96, 768: 64}[D]
    n_chunks = b_per_w // C
    rem = b_per_w - n_chunks * C
    assert rem == 0, (B, D, b_per_w, C)
    mesh = plsc.VectorSubcoreMesh(core_axis_name="c", subcore_axis_name="s")

    @functools.partial(
        pl.kernel, mesh=mesh,
        out_type=jax.ShapeDtypeStruct((B, D), jnp.float32),
        scratch_types=[
            pltpu.VMEM((b_per_w,), jnp.int32),
            pltpu.VMEM((C, D), jnp.float32),
            pltpu.VMEM((C, D), jnp.float32),
            pltpu.SemaphoreType.DMA,
            pltpu.SemaphoreType.DMA,
        ],
    )
    def k(table_hbm, idx_hbm, out_hbm, idx_v, rows0, rows1, sem0, sem1):
        wid = lax.axis_index("s") * 2 + lax.axis_index("c")
        base = wid * b_per_w
        pltpu.sync_copy(idx_hbm.at[pl.ds(base, b_per_w)], idx_v)
        bufs = (rows0, rows1)
        sems = (sem0, sem1)
        copies = [None, None]
        for j in range(n_chunks):
            s = j & 1
            copies[s] = pltpu.async_copy(
                table_hbm.at[idx_v.at[pl.ds(j * C, C)]], bufs[s], sems[s])
            if j > 0:
                p = (j - 1) & 1
                copies[p].wait()
                pltpu.sync_copy(bufs[p], out_hbm.at[pl.ds(base + (j - 1) * C, C)])
        last = (n_chunks - 1) & 1
        copies[last].wait()
        pltpu.sync_copy(bufs[last],
                        out_hbm.at[pl.ds(base + (n_chunks - 1) * C, C)])

    return k(table, idx)


def _level_body(hc3_ref, xwiou_ref, xwf_ref, u_iou_ref, u_f_ref, out_ref):
    h0 = hc3_ref[:, 0, 0:H]
    c0 = hc3_ref[:, 0, H:2 * H]
    h1 = hc3_ref[:, 1, 0:H]
    c1 = hc3_ref[:, 1, H:2 * H]
    xwf = xwf_ref[...]
    u_f = u_f_ref[...]
    f0 = jax.nn.sigmoid(xwf + jnp.dot(h0, u_f, preferred_element_type=jnp.float32))
    f1 = jax.nn.sigmoid(xwf + jnp.dot(h1, u_f, preferred_element_type=jnp.float32))
    fc = f0 * c0 + f1 * c1
    iou = xwiou_ref[...] + jnp.dot(
        h0 + h1, u_iou_ref[...], preferred_element_type=jnp.float32)
    c_new = jax.nn.sigmoid(iou[:, 0:H]) * jnp.tanh(iou[:, 2 * H:3 * H]) + fc
    h_new = jax.nn.sigmoid(iou[:, H:2 * H]) * jnp.tanh(c_new)
    out_ref[:, 0:H] = h_new
    out_ref[:, H:2 * H] = c_new


def _level_call(hc_children, xwiou, xwf, U_iou, U_f):
    np_ = xwiou.shape[0]
    hc3 = hc_children.reshape(np_, 2, 2 * H)
    blk = min(np_, 1536)
    grid = (np_ // blk,)
    return pl.pallas_call(
        _level_body,
        grid=grid,
        in_specs=[
            pl.BlockSpec((blk, 2, 2 * H), lambda i: (i, 0, 0)),
            pl.BlockSpec((blk, 3 * H), lambda i: (i, 0)),
            pl.BlockSpec((blk, H), lambda i: (i, 0)),
            pl.BlockSpec((H, 3 * H), lambda i: (0, 0)),
            pl.BlockSpec((H, H), lambda i: (0, 0)),
        ],
        out_specs=pl.BlockSpec((blk, 2 * H), lambda i: (i, 0)),
        out_shape=jax.ShapeDtypeStruct((np_, 2 * H), jnp.float32),
    )(hc3, xwiou, xwf, U_iou, U_f)


def _head_body(hc_ref, wm_ref, bm_ref, wv_ref, bv_ref, eps_ref,
               z_ref, zm_ref, zlv_ref):
    hroot = hc_ref[:, 0:H]
    zm = jnp.dot(hroot, wm_ref[...], preferred_element_type=jnp.float32) + bm_ref[...]
    zlv = jnp.dot(hroot, wv_ref[...], preferred_element_type=jnp.float32) + bv_ref[...]
    std = jnp.exp(0.5 * zlv)
    z_ref[...] = eps_ref[...] * std + zm
    zm_ref[...] = zm
    zlv_ref[...] = zlv


def _head_call(hc_roots, Wm, bm, Wv, bv, eps):
    o = jax.ShapeDtypeStruct((T, LATENT), jnp.float32)
    return pl.pallas_call(_head_body, out_shape=[o, o, o])(
        hc_roots, Wm, bm, Wv, bv, eps)


def kernel(features, node_order_bottomup, adjacency_list, edge_order_bottomup,
           vocabs, tree_sizes, res_table, leaf_table, W_ih, b_ih, b_hh,
           W_iou, b_iou, U_iou, W_f, b_f, U_f, Wm, bm, Wv, bv):
    b_ihh = (b_ih + b_hh).reshape(1, 4 * H)
    hc_table, xwiou_table, xwf_table = _tables_call(
        leaf_table, res_table, W_ih, b_ihh, W_iou,
        b_iou.reshape(1, 3 * H), W_f, b_f.reshape(1, H))

    ids_leaf = jnp.take(features, _LEAF_PERM).astype(jnp.int32)
    ids_int = jnp.take(features, _INT_PERM).astype(jnp.int32)
    hc = _sc_gather(hc_table, ids_leaf)
    xwiou_int = _sc_gather(xwiou_table, ids_int)
    xwf_int = _sc_gather(xwf_table, ids_int)

    for l in range(DEPTH - 2, -1, -1):
        base = T * (2 ** l - 1)
        cnt = T * 2 ** l
        hc = _level_call(
            hc,
            lax.slice_in_dim(xwiou_int, base, base + cnt, axis=0),
            lax.slice_in_dim(xwf_int, base, base + cnt, axis=0),
            U_iou, U_f)

    eps = jax.random.normal(jax.random.key(42), (T, LATENT), jnp.float32)
    z, zm, zlv = _head_call(hc, Wm, bm.reshape(1, LATENT),
                            Wv, bv.reshape(1, LATENT), eps)
    return (z, zm, zlv)


# speedup vs baseline: 38.6658x; 1.0652x over previous
import functools

import numpy as np
import jax
import jax.numpy as jnp
from jax import lax
from jax.experimental import pallas as pl
from jax.experimental.pallas import tpu as pltpu
from jax.experimental.pallas import tpu_sc as plsc

H = 256
DEPTH = 12
T = 24
S = 2 ** DEPTH - 1
LEAVES = 2 ** (DEPTH - 1)
LATENT = 64
NW = 32
B_GATHER = T * LEAVES


def _build_perms():
    parts = []
    for l in range(DEPTH - 1):
        js = np.arange(2 ** l - 1, 2 ** (l + 1) - 1)
        parts.append((np.arange(T)[:, None] * S + js[None, :]).reshape(-1))
    int_perm = np.concatenate(parts)
    int_perm = np.concatenate(
        [int_perm, np.zeros(B_GATHER - int_perm.size, np.int64)])
    leaf_perm = (np.arange(T)[:, None] * S
                 + np.arange(LEAVES - 1, S)[None, :]).reshape(-1)
    return jnp.asarray(int_perm, jnp.int32), jnp.asarray(leaf_perm, jnp.int32)


_INT_PERM, _LEAF_PERM = _build_perms()


def _tables_body(leaf_t_ref, res_t_ref, w_ih_ref, b_ihh_ref, w_iou_ref,
                 b_iou_ref, w_f_ref, b_f_ref, hc_ref, xw_ref):
    gates = lax.dot_general(
        leaf_t_ref[...], w_ih_ref[...], (((1,), (1,)), ((), ())),
        preferred_element_type=jnp.float32) + b_ihh_ref[...]
    c = jax.nn.sigmoid(gates[:, 0:H]) * jnp.tanh(gates[:, 2 * H:3 * H])
    h = jax.nn.sigmoid(gates[:, 3 * H:4 * H]) * jnp.tanh(c)
    hc_ref[:, 0:H] = h.astype(jnp.bfloat16)
    hc_ref[:, H:2 * H] = c.astype(jnp.bfloat16)
    res = res_t_ref[...]
    xw_ref[:, 0:3 * H] = (jnp.dot(
        res, w_iou_ref[...], preferred_element_type=jnp.float32)
        + b_iou_ref[...]).astype(jnp.bfloat16)
    xw_ref[:, 3 * H:4 * H] = (jnp.dot(
        res, w_f_ref[...], preferred_element_type=jnp.float32)
        + b_f_ref[...]).astype(jnp.bfloat16)


def _tables_call(leaf_table, res_table, W_ih, b_ihh, W_iou, b_iou, W_f, b_f):
    V = leaf_table.shape[0]
    return pl.pallas_call(
        _tables_body,
        out_shape=[
            jax.ShapeDtypeStruct((V, 2 * H), jnp.bfloat16),
            jax.ShapeDtypeStruct((V, 4 * H), jnp.bfloat16),
        ],
    )(leaf_table, res_table, W_ih, b_ihh, W_iou, b_iou, W_f, b_f)


def _sc_gather(table, idx):
    V, D = table.shape
    B = idx.shape[0]
    b_per_w = B // NW
    C = {256: 128, 512: 96}[D]
    n_chunks = b_per_w // C
    assert n_chunks * C == b_per_w, (B, D, b_per_w, C)
    mesh = plsc.VectorSubcoreMesh(core_axis_name="c", subcore_axis_name="s")

    @functools.partial(
        pl.kernel, mesh=mesh,
        out_type=jax.ShapeDtypeStruct((B, D), jnp.int32),
        scratch_types=[
            pltpu.VMEM((b_per_w,), jnp.int32),
            pltpu.VMEM((C, D), jnp.int32),
            pltpu.VMEM((C, D), jnp.int32),
            pltpu.SemaphoreType.DMA,
            pltpu.SemaphoreType.DMA,
        ],
    )
    def k(table_hbm, idx_hbm, out_hbm, idx_v, rows0, rows1, sem0, sem1):
        wid = lax.axis_index("s") * 2 + lax.axis_index("c")
        base = wid * b_per_w
        pltpu.sync_copy(idx_hbm.at[pl.ds(base, b_per_w)], idx_v)
        bufs = (rows0, rows1)
        sems = (sem0, sem1)
        copies = [None, None]
        for j in range(n_chunks):
            s = j & 1
            copies[s] = pltpu.async_copy(
                table_hbm.at[idx_v.at[pl.ds(j * C, C)]], bufs[s], sems[s])
            if j > 0:
                p = (j - 1) & 1
                copies[p].wait()
                pltpu.sync_copy(bufs[p], out_hbm.at[pl.ds(base + (j - 1) * C, C)])
        last = (n_chunks - 1) & 1
        copies[last].wait()
        pltpu.sync_copy(bufs[last],
                        out_hbm.at[pl.ds(base + (n_chunks - 1) * C, C)])

    return k(table, idx)


def _lo(x):
    return lax.bitcast_convert_type(x << 16, jnp.float32)


def _hi(x):
    return lax.bitcast_convert_type(x & jnp.int32(-65536), jnp.float32)


def _level_body(hc3_ref, xw_ref, u_iou_ref, u_f_ref, out_ref, *, leaf):
    if leaf:
        w0 = hc3_ref[:, 0, :]
        w1 = hc3_ref[:, 1, :]
        h0, c0 = _lo(w0), _hi(w0)
        h1, c1 = _lo(w1), _hi(w1)
    else:
        h0 = hc3_ref[:, 0, 0:H]
        c0 = hc3_ref[:, 0, H:2 * H]
        h1 = hc3_ref[:, 1, 0:H]
        c1 = hc3_ref[:, 1, H:2 * H]
    h0b = h0.astype(jnp.bfloat16)
    h1b = h1.astype(jnp.bfloat16)
    xw = xw_ref[...]
    e = _lo(xw)
    o = _hi(xw)
    xwf = o[:, H:2 * H]
    u_f = u_f_ref[...]
    f0 = jax.nn.sigmoid(xwf + jnp.dot(h0b, u_f, preferred_element_type=jnp.float32))
    f1 = jax.nn.sigmoid(xwf + jnp.dot(h1b, u_f, preferred_element_type=jnp.float32))
    fc = f0 * c0 + f1 * c1
    m = jnp.dot(h0b + h1b, u_iou_ref[...], preferred_element_type=jnp.float32)
    c_new = (jax.nn.sigmoid(e[:, 0:H] + m[:, 0:H])
             * jnp.tanh(o[:, 0:H] + m[:, 2 * H:3 * H]) + fc)
    h_new = jax.nn.sigmoid(e[:, H:2 * H] + m[:, H:2 * H]) * jnp.tanh(c_new)
    out_ref[:, 0:H] = h_new
    out_ref[:, H:2 * H] = c_new


def _level_call(hc_children, xw, U_iou, U_f, leaf):
    np_ = xw.shape[0]
    hcw = 2 * H // 2 if leaf else 2 * H
    hc3 = hc_children.reshape(np_, 2, hcw)
    blk = min(np_, 1536)
    grid = (np_ // blk,)
    return pl.pallas_call(
        functools.partial(_level_body, leaf=leaf),
        grid=grid,
        in_specs=[
            pl.BlockSpec((blk, 2, hcw), lambda i: (i, 0, 0)),
            pl.BlockSpec((blk, 2 * H), lambda i: (i, 0)),
            pl.BlockSpec((H, 3 * H), lambda i: (0, 0)),
            pl.BlockSpec((H, H), lambda i: (0, 0)),
        ],
        out_specs=pl.BlockSpec((blk, 2 * H), lambda i: (i, 0)),
        out_shape=jax.ShapeDtypeStruct((np_, 2 * H), jnp.float32),
    )(hc3, xw, U_iou, U_f)


def _head_body(hc_ref, wm_ref, bm_ref, wv_ref, bv_ref, eps_ref,
               z_ref, zm_ref, zlv_ref):
    hroot = hc_ref[:, 0:H]
    zm = jnp.dot(hroot, wm_ref[...], preferred_element_type=jnp.float32) + bm_ref[...]
    zlv = jnp.dot(hroot, wv_ref[...], preferred_element_type=jnp.float32) + bv_ref[...]
    std = jnp.exp(0.5 * zlv)
    z_ref[...] = eps_ref[...] * std + zm
    zm_ref[...] = zm
    zlv_ref[...] = zlv


def _head_call(hc_roots, Wm, bm, Wv, bv, eps):
    o = jax.ShapeDtypeStruct((T, LATENT), jnp.float32)
    return pl.pallas_call(_head_body, out_shape=[o, o, o])(
        hc_roots, Wm, bm, Wv, bv, eps)


def kernel(features, node_order_bottomup, adjacency_list, edge_order_bottomup,
           vocabs, tree_sizes, res_table, leaf_table, W_ih, b_ih, b_hh,
           W_iou, b_iou, U_iou, W_f, b_f, U_f, Wm, bm, Wv, bv):
    b_ihh = (b_ih + b_hh).reshape(1, 4 * H)
    hc_table, xw_table = _tables_call(
        leaf_table, res_table, W_ih, b_ihh, W_iou,
        b_iou.reshape(1, 3 * H), W_f, b_f.reshape(1, H))

    hc_packed = lax.bitcast_convert_type(
        jnp.stack([hc_table[:, 0:H], hc_table[:, H:2 * H]], axis=-1), jnp.int32)
    xw_packed = lax.bitcast_convert_type(
        jnp.stack([xw_table[:, 0:2 * H], xw_table[:, 2 * H:4 * H]], axis=-1),
        jnp.int32)

    ids_leaf = jnp.take(features, _LEAF_PERM).astype(jnp.int32)
    ids_int = jnp.take(features, _INT_PERM).astype(jnp.int32)
    hc = _sc_gather(hc_packed, ids_leaf)
    xw_int = _sc_gather(xw_packed, ids_int)

    U_iou_b = U_iou.astype(jnp.bfloat16)
    U_f_b = U_f.astype(jnp.bfloat16)
    for l in range(DEPTH - 2, -1, -1):
        base = T * (2 ** l - 1)
        cnt = T * 2 ** l
        hc = _level_call(
            hc, lax.slice_in_dim(xw_int, base, base + cnt, axis=0),
            U_iou_b, U_f_b, leaf=(l == DEPTH - 2))

    eps = jax.random.normal(jax.random.key(42), (T, LATENT), jnp.float32)
    z, zm, zlv = _head_call(hc, Wm, bm.reshape(1, LATENT),
                            Wv, bv.reshape(1, LATENT), eps)
    return (z, zm, zlv)


# speedup vs baseline: 90.3425x; 2.3365x over previous
import functools

import numpy as np
import jax
import jax.numpy as jnp
from jax import lax
from jax.experimental import pallas as pl
from jax.experimental.pallas import tpu as pltpu
from jax.experimental.pallas import tpu_sc as plsc

H = 256
DEPTH = 12
T = 24
S = 2 ** DEPTH - 1
LEAVES = 2 ** (DEPTH - 1)
LATENT = 64
NW = 32
B_GATHER = T * LEAVES


XW_BASES = [0, 24, 72, 168, 360, 744, 1512, 3072, 6144, 12288, 24576]


def _brev(r, bits):
    out = np.zeros_like(r)
    for k in range(bits):
        out = (out << 1) | ((r >> k) & 1)
    return out


def _build_perms():
    int_perm = np.zeros(B_GATHER, np.int64)
    t = np.arange(T)
    for l in range(DEPTH - 1):
        r = np.arange(2 ** l)
        j = _brev(r, l)
        rows = XW_BASES[l] + r[:, None] * T + t[None, :]
        nodes = t[None, :] * S + (2 ** l - 1) + j[:, None]
        int_perm[rows.reshape(-1)] = nodes.reshape(-1)
    r = np.arange(LEAVES)
    j = _brev(r, DEPTH - 1)
    leaf_perm = np.zeros(B_GATHER, np.int64)
    rows = r[:, None] * T + t[None, :]
    nodes = t[None, :] * S + (LEAVES - 1) + j[:, None]
    leaf_perm[rows.reshape(-1)] = nodes.reshape(-1)
    return jnp.asarray(int_perm, jnp.int32), jnp.asarray(leaf_perm, jnp.int32)


_INT_PERM, _LEAF_PERM = _build_perms()


def _tables_body(leaf_t_ref, res_t_ref, w_ih_ref, b_ihh_ref, w_iou_ref,
                 b_iou_ref, w_f_ref, b_f_ref, hc_ref, xw_ref):
    gates = lax.dot_general(
        leaf_t_ref[...], w_ih_ref[...], (((1,), (1,)), ((), ())),
        preferred_element_type=jnp.float32) + b_ihh_ref[...]
    c = jax.nn.sigmoid(gates[:, 0:H]) * jnp.tanh(gates[:, 2 * H:3 * H])
    h = jax.nn.sigmoid(gates[:, 3 * H:4 * H]) * jnp.tanh(c)
    hc_ref[:, 0:H] = h.astype(jnp.bfloat16)
    hc_ref[:, H:2 * H] = c.astype(jnp.bfloat16)
    res = res_t_ref[...]
    xw_ref[:, 0:3 * H] = (jnp.dot(
        res, w_iou_ref[...], preferred_element_type=jnp.float32)
        + b_iou_ref[...]).astype(jnp.bfloat16)
    xw_ref[:, 3 * H:4 * H] = (jnp.dot(
        res, w_f_ref[...], preferred_element_type=jnp.float32)
        + b_f_ref[...]).astype(jnp.bfloat16)


def _tables_call(leaf_table, res_table, W_ih, b_ihh, W_iou, b_iou, W_f, b_f):
    V = leaf_table.shape[0]
    return pl.pallas_call(
        _tables_body,
        out_shape=[
            jax.ShapeDtypeStruct((V, 2 * H), jnp.bfloat16),
            jax.ShapeDtypeStruct((V, 4 * H), jnp.bfloat16),
        ],
    )(leaf_table, res_table, W_ih, b_ihh, W_iou, b_iou, W_f, b_f)


def _sc_gather(table, idx):
    V, D = table.shape
    B = idx.shape[0]
    b_per_w = B // NW
    C = {256: 128, 512: 96}[D]
    n_chunks = b_per_w // C
    assert n_chunks * C == b_per_w, (B, D, b_per_w, C)
    mesh = plsc.VectorSubcoreMesh(core_axis_name="c", subcore_axis_name="s")

    @functools.partial(
        pl.kernel, mesh=mesh,
        out_type=jax.ShapeDtypeStruct((B, D), jnp.int32),
        scratch_types=[
            pltpu.VMEM((b_per_w,), jnp.int32),
            pltpu.VMEM((C, D), jnp.int32),
            pltpu.VMEM((C, D), jnp.int32),
            pltpu.SemaphoreType.DMA,
            pltpu.SemaphoreType.DMA,
        ],
    )
    def k(table_hbm, idx_hbm, out_hbm, idx_v, rows0, rows1, sem0, sem1):
        wid = lax.axis_index("s") * 2 + lax.axis_index("c")
        base = wid * b_per_w
        pltpu.sync_copy(idx_hbm.at[pl.ds(base, b_per_w)], idx_v)
        bufs = (rows0, rows1)
        sems = (sem0, sem1)
        copies = [None, None]
        for j in range(n_chunks):
            s = j & 1
            copies[s] = pltpu.async_copy(
                table_hbm.at[idx_v.at[pl.ds(j * C, C)]], bufs[s], sems[s])
            if j > 0:
                p = (j - 1) & 1
                copies[p].wait()
                pltpu.sync_copy(bufs[p], out_hbm.at[pl.ds(base + (j - 1) * C, C)])
        last = (n_chunks - 1) & 1
        copies[last].wait()
        pltpu.sync_copy(bufs[last],
                        out_hbm.at[pl.ds(base + (n_chunks - 1) * C, C)])

    return k(table, idx)


def _lo(x):
    return lax.bitcast_convert_type(x << 16, jnp.float32)


def _hi(x):
    return lax.bitcast_convert_type(x & jnp.int32(-65536), jnp.float32)


def _level_body(hc0_ref, hc1_ref, xw_ref, u_iou_ref, u_f_ref, out_ref, *, leaf):
    if leaf:
        w0 = hc0_ref[...]
        w1 = hc1_ref[...]
        h0, c0 = _lo(w0), _hi(w0)
        h1, c1 = _lo(w1), _hi(w1)
    else:
        h0 = hc0_ref[:, 0:H]
        c0 = hc0_ref[:, H:2 * H]
        h1 = hc1_ref[:, 0:H]
        c1 = hc1_ref[:, H:2 * H]
    h0b = h0.astype(jnp.bfloat16)
    h1b = h1.astype(jnp.bfloat16)
    xw = xw_ref[...]
    e = _lo(xw)
    o = _hi(xw)
    xwf = o[:, H:2 * H]
    u_f = u_f_ref[...]
    f0 = jax.nn.sigmoid(xwf + jnp.dot(h0b, u_f, preferred_element_type=jnp.float32))
    f1 = jax.nn.sigmoid(xwf + jnp.dot(h1b, u_f, preferred_element_type=jnp.float32))
    fc = f0 * c0 + f1 * c1
    m = jnp.dot(h0b + h1b, u_iou_ref[...], preferred_element_type=jnp.float32)
    c_new = (jax.nn.sigmoid(e[:, 0:H] + m[:, 0:H])
             * jnp.tanh(o[:, 0:H] + m[:, 2 * H:3 * H]) + fc)
    h_new = jax.nn.sigmoid(e[:, H:2 * H] + m[:, H:2 * H]) * jnp.tanh(c_new)
    out_ref[:, 0:H] = h_new
    out_ref[:, H:2 * H] = c_new


def _level_call(hc_children, xw, xw_block_off, cnt, U_iou, U_f, leaf):
    hcw = H if leaf else 2 * H
    blk = min(cnt, 1536)
    odd_off = cnt // blk
    return pl.pallas_call(
        functools.partial(_level_body, leaf=leaf),
        grid=(cnt // blk,),
        in_specs=[
            pl.BlockSpec((blk, hcw), lambda i: (i, 0)),
            pl.BlockSpec((blk, hcw), lambda i: (i + odd_off, 0)),
            pl.BlockSpec((blk, 2 * H), lambda i: (i + xw_block_off, 0)),
            pl.BlockSpec((H, 3 * H), lambda i: (0, 0)),
            pl.BlockSpec((H, H), lambda i: (0, 0)),
        ],
        out_specs=pl.BlockSpec((blk, 2 * H), lambda i: (i, 0)),
        out_shape=jax.ShapeDtypeStruct((cnt, 2 * H), jnp.float32),
    )(hc_children, hc_children, xw, U_iou, U_f)


def _head_body(hc_ref, wm_ref, bm_ref, wv_ref, bv_ref, eps_ref,
               z_ref, zm_ref, zlv_ref):
    hroot = hc_ref[:, 0:H]
    zm = jnp.dot(hroot, wm_ref[...], preferred_element_type=jnp.float32) + bm_ref[...]
    zlv = jnp.dot(hroot, wv_ref[...], preferred_element_type=jnp.float32) + bv_ref[...]
    std = jnp.exp(0.5 * zlv)
    z_ref[...] = eps_ref[...] * std + zm
    zm_ref[...] = zm
    zlv_ref[...] = zlv


def _head_call(hc_roots, Wm, bm, Wv, bv, eps):
    o = jax.ShapeDtypeStruct((T, LATENT), jnp.float32)
    return pl.pallas_call(_head_body, out_shape=[o, o, o])(
        hc_roots, Wm, bm, Wv, bv, eps)


def kernel(features, node_order_bottomup, adjacency_list, edge_order_bottomup,
           vocabs, tree_sizes, res_table, leaf_table, W_ih, b_ih, b_hh,
           W_iou, b_iou, U_iou, W_f, b_f, U_f, Wm, bm, Wv, bv):
    b_ihh = (b_ih + b_hh).reshape(1, 4 * H)
    hc_table, xw_table = _tables_call(
        leaf_table, res_table, W_ih, b_ihh, W_iou,
        b_iou.reshape(1, 3 * H), W_f, b_f.reshape(1, H))

    hc_packed = lax.bitcast_convert_type(
        jnp.stack([hc_table[:, 0:H], hc_table[:, H:2 * H]], axis=-1), jnp.int32)
    xw_packed = lax.bitcast_convert_type(
        jnp.stack([xw_table[:, 0:2 * H], xw_table[:, 2 * H:4 * H]], axis=-1),
        jnp.int32)

    ids_leaf = jnp.take(features, _LEAF_PERM).astype(jnp.int32)
    ids_int = jnp.take(features, _INT_PERM).astype(jnp.int32)
    hc = _sc_gather(hc_packed, ids_leaf)
    xw_int = _sc_gather(xw_packed, ids_int)

    U_iou_b = U_iou.astype(jnp.bfloat16)
    U_f_b = U_f.astype(jnp.bfloat16)
    for l in range(DEPTH - 2, -1, -1):
        base = XW_BASES[l]
        cnt = T * 2 ** l
        blk = min(cnt, 1536)
        if base % blk == 0:
            xw_l, off = xw_int, base // blk
        else:
            xw_l, off = lax.slice_in_dim(xw_int, base, base + cnt, axis=0), 0
        hc = _level_call(hc, xw_l, off, cnt, U_iou_b, U_f_b,
                         leaf=(l == DEPTH - 2))

    eps = jax.random.normal(jax.random.key(42), (T, LATENT), jnp.float32)
    z, zm, zlv = _head_call(hc, Wm, bm.reshape(1, LATENT),
                            Wv, bv.reshape(1, LATENT), eps)
    return (z, zm, zlv)


# speedup vs baseline: 102.4771x; 1.1343x over previous
import functools

import numpy as np
import jax
import jax.numpy as jnp
from jax import lax
from jax.experimental import pallas as pl
from jax.experimental.pallas import tpu as pltpu
from jax.experimental.pallas import tpu_sc as plsc

H = 256
DEPTH = 12
T = 24
S = 2 ** DEPTH - 1
LEAVES = 2 ** (DEPTH - 1)
LATENT = 64
NW = 32
B_GATHER = T * LEAVES


XW_BASES = [0, 24, 72, 168, 360, 744, 1512, 3072, 6144, 12288, 24576]


def _brev(r, bits):
    out = np.zeros_like(r)
    for k in range(bits):
        out = (out << 1) | ((r >> k) & 1)
    return out


def _build_perms():
    int_perm = np.zeros(B_GATHER, np.int64)
    t = np.arange(T)
    for l in range(DEPTH - 1):
        r = np.arange(2 ** l)
        j = _brev(r, l)
        rows = XW_BASES[l] + r[:, None] * T + t[None, :]
        nodes = t[None, :] * S + (2 ** l - 1) + j[:, None]
        int_perm[rows.reshape(-1)] = nodes.reshape(-1)
    r = np.arange(LEAVES)
    j = _brev(r, DEPTH - 1)
    leaf_perm = np.zeros(B_GATHER, np.int64)
    rows = r[:, None] * T + t[None, :]
    nodes = t[None, :] * S + (LEAVES - 1) + j[:, None]
    leaf_perm[rows.reshape(-1)] = nodes.reshape(-1)
    return jnp.asarray(int_perm, jnp.int32), jnp.asarray(leaf_perm, jnp.int32)


_INT_PERM, _LEAF_PERM = _build_perms()


def _tables_body(leaf_t_ref, res_t_ref, w_ih_ref, b_ihh_ref, w_iou_ref,
                 b_iou_ref, w_f_ref, b_f_ref, hc_ref, xw_ref):
    gates = lax.dot_general(
        leaf_t_ref[...], w_ih_ref[...], (((1,), (1,)), ((), ())),
        preferred_element_type=jnp.float32) + b_ihh_ref[...]
    c = jax.nn.sigmoid(gates[:, 0:H]) * jnp.tanh(gates[:, 2 * H:3 * H])
    h = jax.nn.sigmoid(gates[:, 3 * H:4 * H]) * jnp.tanh(c)
    hc_ref[:, 0:H] = h.astype(jnp.bfloat16)
    hc_ref[:, H:2 * H] = c.astype(jnp.bfloat16)
    res = res_t_ref[...]
    xw_ref[:, 0:3 * H] = (jnp.dot(
        res, w_iou_ref[...], preferred_element_type=jnp.float32)
        + b_iou_ref[...]).astype(jnp.bfloat16)
    xw_ref[:, 3 * H:4 * H] = (jnp.dot(
        res, w_f_ref[...], preferred_element_type=jnp.float32)
        + b_f_ref[...]).astype(jnp.bfloat16)


def _tables_call(leaf_table, res_table, W_ih, b_ihh, W_iou, b_iou, W_f, b_f):
    V = leaf_table.shape[0]
    return pl.pallas_call(
        _tables_body,
        out_shape=[
            jax.ShapeDtypeStruct((V, 2 * H), jnp.bfloat16),
            jax.ShapeDtypeStruct((V, 4 * H), jnp.bfloat16),
        ],
    )(leaf_table, res_table, W_ih, b_ihh, W_iou, b_iou, W_f, b_f)


def _sc_gather(table, idx):
    V, D = table.shape
    B = idx.shape[0]
    b_per_w = B // NW
    C = {256: 128, 512: 96}[D]
    n_chunks = b_per_w // C
    assert n_chunks * C == b_per_w, (B, D, b_per_w, C)
    mesh = plsc.VectorSubcoreMesh(core_axis_name="c", subcore_axis_name="s")

    @functools.partial(
        pl.kernel, mesh=mesh,
        out_type=jax.ShapeDtypeStruct((B, D), jnp.int32),
        scratch_types=[
            pltpu.VMEM((b_per_w,), jnp.int32),
            pltpu.VMEM((C, D), jnp.int32),
            pltpu.VMEM((C, D), jnp.int32),
            pltpu.SemaphoreType.DMA,
            pltpu.SemaphoreType.DMA,
        ],
    )
    def k(table_hbm, idx_hbm, out_hbm, idx_v, rows0, rows1, sem0, sem1):
        wid = lax.axis_index("s") * 2 + lax.axis_index("c")
        base = wid * b_per_w
        pltpu.sync_copy(idx_hbm.at[pl.ds(base, b_per_w)], idx_v)
        bufs = (rows0, rows1)
        sems = (sem0, sem1)
        copies = [None, None]
        for j in range(n_chunks):
            s = j & 1
            copies[s] = pltpu.async_copy(
                table_hbm.at[idx_v.at[pl.ds(j * C, C)]], bufs[s], sems[s])
            if j > 0:
                p = (j - 1) & 1
                copies[p].wait()
                pltpu.sync_copy(bufs[p], out_hbm.at[pl.ds(base + (j - 1) * C, C)])
        last = (n_chunks - 1) & 1
        copies[last].wait()
        pltpu.sync_copy(bufs[last],
                        out_hbm.at[pl.ds(base + (n_chunks - 1) * C, C)])

    return k(table, idx)


def _lo(x):
    return lax.bitcast_convert_type(x << 16, jnp.float32)


def _hi(x):
    return lax.bitcast_convert_type(x & jnp.int32(-65536), jnp.float32)


def _pack(h, c):
    hb = lax.bitcast_convert_type(
        h.astype(jnp.bfloat16).astype(jnp.float32), jnp.int32)
    cb = lax.bitcast_convert_type(
        c.astype(jnp.bfloat16).astype(jnp.float32), jnp.int32)
    return lax.shift_right_logical(hb, 16) | cb


def _cell(w0, w1, xw, u_iou, u_f):
    h0, c0 = _lo(w0), _hi(w0)
    h1, c1 = _lo(w1), _hi(w1)
    h0b = h0.astype(jnp.bfloat16)
    h1b = h1.astype(jnp.bfloat16)
    e = _lo(xw)
    o = _hi(xw)
    xwf = o[:, H:2 * H]
    f0 = jax.nn.sigmoid(xwf + jnp.dot(h0b, u_f, preferred_element_type=jnp.float32))
    f1 = jax.nn.sigmoid(xwf + jnp.dot(h1b, u_f, preferred_element_type=jnp.float32))
    fc = f0 * c0 + f1 * c1
    m = jnp.dot(h0b + h1b, u_iou, preferred_element_type=jnp.float32)
    c_new = (jax.nn.sigmoid(e[:, 0:H] + m[:, 0:H])
             * jnp.tanh(o[:, 0:H] + m[:, 2 * H:3 * H]) + fc)
    h_new = jax.nn.sigmoid(e[:, H:2 * H] + m[:, H:2 * H]) * jnp.tanh(c_new)
    return h_new, c_new


def _level_body(hc0_ref, hc1_ref, xw_ref, u_iou_ref, u_f_ref, out_ref):
    h_new, c_new = _cell(hc0_ref[...], hc1_ref[...], xw_ref[...],
                         u_iou_ref[...], u_f_ref[...])
    out_ref[...] = _pack(h_new, c_new)


def _level_call(hc_children, xw, xw_block_off, cnt, U_iou, U_f):
    blk = min(cnt, 1536)
    odd_off = cnt // blk
    return pl.pallas_call(
        _level_body,
        grid=(cnt // blk,),
        in_specs=[
            pl.BlockSpec((blk, H), lambda i: (i, 0)),
            pl.BlockSpec((blk, H), lambda i: (i + odd_off, 0)),
            pl.BlockSpec((blk, 2 * H), lambda i: (i + xw_block_off, 0)),
            pl.BlockSpec((H, 3 * H), lambda i: (0, 0)),
            pl.BlockSpec((H, H), lambda i: (0, 0)),
        ],
        out_specs=pl.BlockSpec((blk, H), lambda i: (i, 0)),
        out_shape=jax.ShapeDtypeStruct((cnt, H), jnp.int32),
    )(hc_children, hc_children, xw, U_iou, U_f)


def _top_body(hc7_ref, xw_ref, u_iou_ref, u_f_ref, wm_ref, bm_ref, wv_ref,
              bv_ref, eps_ref, z_ref, zm_ref, zlv_ref):
    u_iou = u_iou_ref[...]
    u_f = u_f_ref[...]
    hc = hc7_ref[...]
    h_new = None
    for l in range(6, -1, -1):
        cnt = T * 2 ** l
        w0 = hc[0:cnt]
        w1 = hc[cnt:2 * cnt]
        xw = xw_ref[XW_BASES[l]:XW_BASES[l] + cnt]
        h_new, c_new = _cell(w0, w1, xw, u_iou, u_f)
        if l > 0:
            hc = _pack(h_new, c_new)
    hroot = h_new
    zm = jnp.dot(hroot, wm_ref[...], preferred_element_type=jnp.float32) + bm_ref[...]
    zlv = jnp.dot(hroot, wv_ref[...], preferred_element_type=jnp.float32) + bv_ref[...]
    std = jnp.exp(0.5 * zlv)
    z_ref[...] = eps_ref[...] * std + zm
    zm_ref[...] = zm
    zlv_ref[...] = zlv


def _top_call(hc7, xw_small, U_iou, U_f, Wm, bm, Wv, bv, eps):
    o = jax.ShapeDtypeStruct((T, LATENT), jnp.float32)
    return pl.pallas_call(_top_body, out_shape=[o, o, o])(
        hc7, xw_small, U_iou, U_f, Wm, bm, Wv, bv, eps)


def kernel(features, node_order_bottomup, adjacency_list, edge_order_bottomup,
           vocabs, tree_sizes, res_table, leaf_table, W_ih, b_ih, b_hh,
           W_iou, b_iou, U_iou, W_f, b_f, U_f, Wm, bm, Wv, bv):
    b_ihh = (b_ih + b_hh).reshape(1, 4 * H)
    hc_table, xw_table = _tables_call(
        leaf_table, res_table, W_ih, b_ihh, W_iou,
        b_iou.reshape(1, 3 * H), W_f, b_f.reshape(1, H))

    hc_packed = lax.bitcast_convert_type(
        jnp.stack([hc_table[:, 0:H], hc_table[:, H:2 * H]], axis=-1), jnp.int32)
    xw_packed = lax.bitcast_convert_type(
        jnp.stack([xw_table[:, 0:2 * H], xw_table[:, 2 * H:4 * H]], axis=-1),
        jnp.int32)

    ids_leaf = jnp.take(features, _LEAF_PERM).astype(jnp.int32)
    ids_int = jnp.take(features, _INT_PERM).astype(jnp.int32)
    hc = _sc_gather(hc_packed, ids_leaf)
    xw_int = _sc_gather(xw_packed, ids_int)

    U_iou_b = U_iou.astype(jnp.bfloat16)
    U_f_b = U_f.astype(jnp.bfloat16)
    for l in range(DEPTH - 2, 6, -1):
        cnt = T * 2 ** l
        hc = _level_call(hc, xw_int, XW_BASES[l] // 1536, cnt, U_iou_b, U_f_b)

    eps = jax.random.normal(jax.random.key(42), (T, LATENT), jnp.float32)
    xw_small = lax.slice_in_dim(xw_int, 0, XW_BASES[7], axis=0)
    z, zm, zlv = _top_call(hc, xw_small, U_iou_b, U_f_b,
                           Wm, bm.reshape(1, LATENT),
                           Wv, bv.reshape(1, LATENT), eps)
    return (z, zm, zlv)


# speedup vs baseline: 126.4098x; 1.2335x over previous
import functools

import numpy as np
import jax
import jax.numpy as jnp
from jax import lax
from jax.experimental import pallas as pl
from jax.experimental.pallas import tpu as pltpu
from jax.experimental.pallas import tpu_sc as plsc

H = 256
DEPTH = 12
T = 24
S = 2 ** DEPTH - 1
LEAVES = 2 ** (DEPTH - 1)
LATENT = 64
NW = 32
B_GATHER = T * LEAVES


XW_BASES = [0, 24, 72, 168, 360, 744, 1512, 3072, 6144, 12288, 24576]


def _brev(r, bits):
    out = np.zeros_like(r)
    for k in range(bits):
        out = (out << 1) | ((r >> k) & 1)
    return out


def _build_perms():
    int_perm = np.zeros(B_GATHER, np.int64)
    t = np.arange(T)
    for l in range(DEPTH - 1):
        r = np.arange(2 ** l)
        j = _brev(r, l)
        rows = XW_BASES[l] + r[:, None] * T + t[None, :]
        nodes = t[None, :] * S + (2 ** l - 1) + j[:, None]
        int_perm[rows.reshape(-1)] = nodes.reshape(-1)
    r = np.arange(LEAVES)
    j = _brev(r, DEPTH - 1)
    leaf_perm = np.zeros(B_GATHER, np.int64)
    rows = r[:, None] * T + t[None, :]
    nodes = t[None, :] * S + (LEAVES - 1) + j[:, None]
    leaf_perm[rows.reshape(-1)] = nodes.reshape(-1)
    return jnp.asarray(int_perm, jnp.int32), jnp.asarray(leaf_perm, jnp.int32)


_INT_PERM, _LEAF_PERM = _build_perms()


def _tables_body(leaf_t_ref, w_ih_ref, b_ihh_ref, hc_ref):
    gates = lax.dot_general(
        leaf_t_ref[...], w_ih_ref[...], (((1,), (1,)), ((), ())),
        preferred_element_type=jnp.float32) + b_ihh_ref[...]
    c = jax.nn.sigmoid(gates[:, 0:H]) * jnp.tanh(gates[:, 2 * H:3 * H])
    h = jax.nn.sigmoid(gates[:, 3 * H:4 * H]) * jnp.tanh(c)
    hc_ref[:, 0:H] = h.astype(jnp.bfloat16)
    hc_ref[:, H:2 * H] = c.astype(jnp.bfloat16)


def _tables_call(leaf_table, W_ih, b_ihh):
    V = leaf_table.shape[0]
    return pl.pallas_call(
        _tables_body,
        out_shape=jax.ShapeDtypeStruct((V, 2 * H), jnp.bfloat16),
    )(leaf_table, W_ih, b_ihh)


def _sc_gather(table, idx):
    V, D = table.shape
    B = idx.shape[0]
    b_per_w = B // NW
    C = {128: 128, 256: 128, 512: 96}[D]
    n_chunks = b_per_w // C
    assert n_chunks * C == b_per_w, (B, D, b_per_w, C)
    mesh = plsc.VectorSubcoreMesh(core_axis_name="c", subcore_axis_name="s")

    @functools.partial(
        pl.kernel, mesh=mesh,
        out_type=jax.ShapeDtypeStruct((B, D), table.dtype),
        scratch_types=[
            pltpu.VMEM((b_per_w,), jnp.int32),
            pltpu.VMEM((C, D), table.dtype),
            pltpu.VMEM((C, D), table.dtype),
            pltpu.SemaphoreType.DMA,
            pltpu.SemaphoreType.DMA,
        ],
    )
    def k(table_hbm, idx_hbm, out_hbm, idx_v, rows0, rows1, sem0, sem1):
        wid = lax.axis_index("s") * 2 + lax.axis_index("c")
        base = wid * b_per_w
        pltpu.sync_copy(idx_hbm.at[pl.ds(base, b_per_w)], idx_v)
        bufs = (rows0, rows1)
        sems = (sem0, sem1)
        copies = [None, None]
        for j in range(n_chunks):
            s = j & 1
            copies[s] = pltpu.async_copy(
                table_hbm.at[idx_v.at[pl.ds(j * C, C)]], bufs[s], sems[s])
            if j > 0:
                p = (j - 1) & 1
                copies[p].wait()
                pltpu.sync_copy(bufs[p], out_hbm.at[pl.ds(base + (j - 1) * C, C)])
        last = (n_chunks - 1) & 1
        copies[last].wait()
        pltpu.sync_copy(bufs[last],
                        out_hbm.at[pl.ds(base + (n_chunks - 1) * C, C)])

    return k(table, idx)


def _lo(x):
    return lax.bitcast_convert_type(x << 16, jnp.float32)


def _hi(x):
    return lax.bitcast_convert_type(x & jnp.int32(-65536), jnp.float32)


def _pack(h, c):
    hb = lax.bitcast_convert_type(
        h.astype(jnp.bfloat16).astype(jnp.float32), jnp.int32)
    cb = lax.bitcast_convert_type(
        c.astype(jnp.bfloat16).astype(jnp.float32), jnp.int32)
    return lax.shift_right_logical(hb, 16) | cb


def _cell(w0, w1, emb, u_iou, u_f, w_iou, w_f, b_iou, b_f):
    h0, c0 = _lo(w0), _hi(w0)
    h1, c1 = _lo(w1), _hi(w1)
    h0b = h0.astype(jnp.bfloat16)
    h1b = h1.astype(jnp.bfloat16)
    x = emb.astype(jnp.bfloat16)
    xwf = jnp.dot(x, w_f, preferred_element_type=jnp.float32) + b_f
    e = jnp.dot(x, w_iou, preferred_element_type=jnp.float32) + b_iou
    f0 = jax.nn.sigmoid(xwf + jnp.dot(h0b, u_f, preferred_element_type=jnp.float32))
    f1 = jax.nn.sigmoid(xwf + jnp.dot(h1b, u_f, preferred_element_type=jnp.float32))
    fc = f0 * c0 + f1 * c1
    m = jnp.dot(h0b + h1b, u_iou, preferred_element_type=jnp.float32)
    c_new = (jax.nn.sigmoid(e[:, 0:H] + m[:, 0:H])
             * jnp.tanh(e[:, 2 * H:3 * H] + m[:, 2 * H:3 * H]) + fc)
    h_new = (jax.nn.sigmoid(e[:, H:2 * H] + m[:, H:2 * H])
             * jnp.tanh(c_new))
    return h_new, c_new


def _level_body(hc0_ref, hc1_ref, emb_ref, u_iou_ref, u_f_ref, w_iou_ref,
                w_f_ref, b_iou_ref, b_f_ref, out_ref):
    h_new, c_new = _cell(hc0_ref[...], hc1_ref[...], emb_ref[...],
                         u_iou_ref[...], u_f_ref[...], w_iou_ref[...],
                         w_f_ref[...], b_iou_ref[...], b_f_ref[...])
    out_ref[...] = _pack(h_new, c_new)


EMBW = 128


def _level_call(hc_children, emb, emb_block_off, cnt, consts):
    blk = min(cnt, 1536)
    odd_off = cnt // blk
    return pl.pallas_call(
        _level_body,
        grid=(cnt // blk,),
        in_specs=[
            pl.BlockSpec((blk, H), lambda i: (i, 0)),
            pl.BlockSpec((blk, H), lambda i: (i + odd_off, 0)),
            pl.BlockSpec((blk, EMBW), lambda i: (i + emb_block_off, 0)),
            pl.BlockSpec((H, 3 * H), lambda i: (0, 0)),
            pl.BlockSpec((H, H), lambda i: (0, 0)),
            pl.BlockSpec((EMBW, 3 * H), lambda i: (0, 0)),
            pl.BlockSpec((EMBW, H), lambda i: (0, 0)),
            pl.BlockSpec((1, 3 * H), lambda i: (0, 0)),
            pl.BlockSpec((1, H), lambda i: (0, 0)),
        ],
        out_specs=pl.BlockSpec((blk, H), lambda i: (i, 0)),
        out_shape=jax.ShapeDtypeStruct((cnt, H), jnp.int32),
    )(hc_children, hc_children, emb, *consts)


def _top_body(hc7_ref, emb_ref, u_iou_ref, u_f_ref, w_iou_ref, w_f_ref,
              b_iou_ref, b_f_ref, wm_ref, bm_ref, wv_ref, bv_ref, eps_ref,
              z_ref, zm_ref, zlv_ref):
    consts = (u_iou_ref[...], u_f_ref[...], w_iou_ref[...], w_f_ref[...],
              b_iou_ref[...], b_f_ref[...])
    hc = hc7_ref[...]
    h_new = None
    for l in range(6, -1, -1):
        cnt = T * 2 ** l
        w0 = hc[0:cnt]
        w1 = hc[cnt:2 * cnt]
        emb = emb_ref[XW_BASES[l]:XW_BASES[l] + cnt]
        h_new, c_new = _cell(w0, w1, emb, *consts)
        if l > 0:
            hc = _pack(h_new, c_new)
    hroot = h_new
    zm = jnp.dot(hroot, wm_ref[...], preferred_element_type=jnp.float32) + bm_ref[...]
    zlv = jnp.dot(hroot, wv_ref[...], preferred_element_type=jnp.float32) + bv_ref[...]
    std = jnp.exp(0.5 * zlv)
    z_ref[...] = eps_ref[...] * std + zm
    zm_ref[...] = zm
    zlv_ref[...] = zlv


def _top_call(hc7, emb_small, consts, Wm, bm, Wv, bv, eps):
    o = jax.ShapeDtypeStruct((T, LATENT), jnp.float32)
    return pl.pallas_call(_top_body, out_shape=[o, o, o])(
        hc7, emb_small, *consts, Wm, bm, Wv, bv, eps)


def kernel(features, node_order_bottomup, adjacency_list, edge_order_bottomup,
           vocabs, tree_sizes, res_table, leaf_table, W_ih, b_ih, b_hh,
           W_iou, b_iou, U_iou, W_f, b_f, U_f, Wm, bm, Wv, bv):
    b_ihh = (b_ih + b_hh).reshape(1, 4 * H)
    hc_table = _tables_call(leaf_table, W_ih, b_ihh)

    hc_packed = lax.bitcast_convert_type(
        jnp.stack([hc_table[:, 0:H], hc_table[:, H:2 * H]], axis=-1), jnp.int32)
    ids_leaf = jnp.take(features, _LEAF_PERM).astype(jnp.int32)
    ids_int = jnp.take(features, _INT_PERM).astype(jnp.int32)
    hc = _sc_gather(hc_packed, ids_leaf)
    emb_int = _sc_gather(res_table, ids_int)

    consts = (U_iou.astype(jnp.bfloat16), U_f.astype(jnp.bfloat16),
              W_iou.astype(jnp.bfloat16), W_f.astype(jnp.bfloat16),
              b_iou.reshape(1, 3 * H), b_f.reshape(1, H))
    for l in range(DEPTH - 2, 6, -1):
        cnt = T * 2 ** l
        hc = _level_call(hc, emb_int, XW_BASES[l] // 1536, cnt, consts)

    eps = jax.random.normal(jax.random.key(42), (T, LATENT), jnp.float32)
    emb_small = lax.slice_in_dim(emb_int, 0, XW_BASES[7], axis=0)
    z, zm, zlv = _top_call(hc, emb_small, consts,
                           Wm, bm.reshape(1, LATENT),
                           Wv, bv.reshape(1, LATENT), eps)
    return (z, zm, zlv)
